# baseline (speedup 1.0000x reference)
import jax
import jax.numpy as jnp
from jax.experimental import pallas as pl
from jax.experimental.pallas import tpu as pltpu

_B, _C, _L = 32, 64, 8192
_K = 4
_OC = _C * _K
_SE = _L // 2
_LOUT = 4099


def _sp_kernel(xe_ref, w_ref, o_ref):
    xe = jnp.pad(xe_ref[0], ((0, 0), (3, 3)))
    xcat = jnp.concatenate(
        [xe[:, k:k + _LOUT] for k in range(_K)], axis=0)
    o_ref[0] = jax.lax.dot_general(
        w_ref[...], xcat, (((1,), (0,)), ((), ())),
        preferred_element_type=jnp.float32)


def kernel(x, weight):
    xe = x[:, :, ::2]
    wbig = jnp.transpose(weight, (0, 2, 1)).reshape(_OC, _OC)
    return pl.pallas_call(
        _sp_kernel,
        grid=(_B,),
        in_specs=[
            pl.BlockSpec((1, _C, _SE), lambda b: (b, 0, 0)),
            pl.BlockSpec((_OC, _OC), lambda b: (0, 0)),
        ],
        out_specs=pl.BlockSpec((1, _OC, _LOUT), lambda b: (b, 0, 0)),
        out_shape=jax.ShapeDtypeStruct((_B, _OC, _LOUT), jnp.float32),
        compiler_params=pltpu.CompilerParams(
            dimension_semantics=("parallel",),
            vmem_limit_bytes=100 * 1024 * 1024,
        ),
    )(xe, wbig)


# speedup vs baseline: 4.6361x; 4.6361x over previous
import jax
import jax.numpy as jnp
from jax.experimental import pallas as pl
from jax.experimental.pallas import tpu as pltpu

_B, _C, _L = 32, 64, 8192
_K = 4
_OC = _C * _K
_SE = _L // 2
_LOUT = 4099
_NCH = _SE // 128


def _sp_kernel(x_ref, s_ref, w_ref, o_ref):
    x3 = x_ref[0]
    xf = x3.reshape(_C * _NCH, 256)
    e2 = jax.lax.dot_general(
        xf, s_ref[...], (((1,), (0,)), ((), ())),
        preferred_element_type=jnp.float32)
    e = e2.reshape(_C, _NCH, 128)
    ep = jnp.pad(e, ((0, 0), (1, 1), (0, 0)))
    w = w_ref[...]
    for m in range(_NCH + 1):
        left = ep[:, m, :]
        cur = ep[:, m + 1, :]
        parts = []
        for k in range(_K):
            if k < 3:
                parts.append(jnp.concatenate(
                    [left[:, 125 + k:], cur[:, :125 + k]], axis=1))
            else:
                parts.append(cur)
        xcat = jnp.concatenate(parts, axis=0)
        o = jax.lax.dot_general(
            w, xcat, (((1,), (0,)), ((), ())),
            preferred_element_type=jnp.float32)
        if m < _NCH:
            o_ref[0, :, m * 128:(m + 1) * 128] = o
        else:
            o_ref[0, :, _NCH * 128:_LOUT] = o[:, :_LOUT - _NCH * 128]


def kernel(x, weight):
    xr = x.reshape(_B, _C, _NCH, 256)
    s = (jnp.arange(256)[:, None] == 2 * jnp.arange(128)[None, :]
         ).astype(jnp.float32)
    wbig = jnp.transpose(weight, (0, 2, 1)).reshape(_OC, _OC)
    return pl.pallas_call(
        _sp_kernel,
        grid=(_B,),
        in_specs=[
            pl.BlockSpec((1, _C, _NCH, 256), lambda b: (b, 0, 0, 0)),
            pl.BlockSpec((256, 128), lambda b: (0, 0)),
            pl.BlockSpec((_OC, _OC), lambda b: (0, 0)),
        ],
        out_specs=pl.BlockSpec((1, _OC, _LOUT), lambda b: (b, 0, 0)),
        out_shape=jax.ShapeDtypeStruct((_B, _OC, _LOUT), jnp.float32),
        compiler_params=pltpu.CompilerParams(
            dimension_semantics=("parallel",),
            vmem_limit_bytes=100 * 1024 * 1024,
        ),
    )(xr, s, wbig)


# speedup vs baseline: 4.9232x; 1.0619x over previous
import jax
import jax.numpy as jnp
from jax.experimental import pallas as pl
from jax.experimental.pallas import tpu as pltpu

_B, _C, _L = 32, 64, 8192
_K = 4
_OC = _C * _K
_LOUT = 4099
_NCH = 32


def _sp_kernel(x_ref, t_ref, w_ref, o_ref, zs_ref):
    x = x_ref[0]
    tall = t_ref[...]
    z256 = jnp.zeros((_C, 256), dtype=jnp.float32)
    for m in range(_NCH + 1):
        if m == 0:
            v = jnp.concatenate([z256, x[:, :256]], axis=1)
        elif m == _NCH:
            v = jnp.concatenate([x[:, _L - 256:], z256], axis=1)
        else:
            v = x[:, 256 * (m - 1):256 * (m + 1)]
        z = jax.lax.dot_general(
            v, tall, (((1,), (0,)), ((), ())),
            preferred_element_type=jnp.float32)
        for k in range(_K):
            zs_ref[64 * k:64 * (k + 1), 128 * m:128 * (m + 1)] = (
                z[:, 128 * k:128 * (k + 1)])
    w = w_ref[...]
    for m in range(_NCH + 1):
        o = jax.lax.dot_general(
            w, zs_ref[:, 128 * m:128 * (m + 1)], (((1,), (0,)), ((), ())),
            preferred_element_type=jnp.float32)
        if m < _NCH:
            o_ref[0, :, m * 128:(m + 1) * 128] = o
        else:
            o_ref[0, :, _NCH * 128:_LOUT] = o[:, :_LOUT - _NCH * 128]


def kernel(x, weight):
    cols = jnp.arange(512)
    qsel = 250 + 2 * (cols % 128) + 2 * (cols // 128)
    tall = (jnp.arange(512)[:, None] == qsel[None, :]).astype(jnp.float32)
    wbig = jnp.transpose(weight, (0, 2, 1)).reshape(_OC, _OC)
    return pl.pallas_call(
        _sp_kernel,
        grid=(_B,),
        in_specs=[
            pl.BlockSpec((1, _C, _L), lambda b: (b, 0, 0)),
            pl.BlockSpec((512, 512), lambda b: (0, 0)),
            pl.BlockSpec((_OC, _OC), lambda b: (0, 0)),
        ],
        out_specs=pl.BlockSpec((1, _OC, _LOUT), lambda b: (b, 0, 0)),
        out_shape=jax.ShapeDtypeStruct((_B, _OC, _LOUT), jnp.float32),
        scratch_shapes=[pltpu.VMEM((_OC, 128 * (_NCH + 1)), jnp.float32)],
        compiler_params=pltpu.CompilerParams(
            dimension_semantics=("parallel",),
            vmem_limit_bytes=100 * 1024 * 1024,
        ),
    )(x, tall, wbig)
